# baseline (speedup 1.0000x reference)
import jax
import jax.numpy as jnp
from jax.experimental import pallas as pl
from jax.experimental.pallas import tpu as pltpu

_B, _T, _C, _H, _W = 8, 16, 64, 64, 64
_T2 = _T // 2
_G = 16
_N = _B * _T2


def _body(x_ref, v_ref, o_ref, in_stage, out_stage, in_sems, out_sems, bulk_sem):
    even = (jax.lax.broadcasted_iota(jnp.int32, (_G * 32, 128), 1) % 2) == 0
    row = jax.lax.broadcasted_iota(jnp.int32, (64, 128), 0)
    col = jax.lax.broadcasted_iota(jnp.int32, (64, 128), 1)
    expand = jnp.where(col // 2 == row, 1.0, 0.0).astype(jnp.float32)

    def in_copy(i):
        b, j = i // _T2, i % _T2
        return pltpu.make_async_copy(
            x_ref.at[b, 2 * j, :, 0], in_stage.at[i], in_sems.at[i])

    def out_copy(i):
        b, j = i // _T2, i % _T2
        return pltpu.make_async_copy(
            out_stage.at[i], o_ref.at[b, 2 * j, :, 0], out_sems.at[i])

    def bulk_odd(i):
        b, j = i // _T2, i % _T2
        return pltpu.make_async_copy(
            x_ref.at[b, 2 * j + 1], o_ref.at[b, 2 * j + 1], bulk_sem)

    def bulk_rest(i):
        b, j = i // _T2, i % _T2
        return pltpu.make_async_copy(
            x_ref.at[b, 2 * j, :, pl.ds(1, 3)],
            o_ref.at[b, 2 * j, :, pl.ds(1, 3)], bulk_sem)

    def issue(i, carry):
        in_copy(i).start()
        bulk_odd(i).start()
        bulk_rest(i).start()
        return carry

    jax.lax.fori_loop(0, _N, issue, 0, unroll=False)

    def compute(i, carry):
        b, j = i // _T2, i % _T2
        in_copy(i).wait()
        x = in_stage[i].reshape(_G * 32, 128)
        v = v_ref[b, j].reshape(_G * 32, 64)
        vexp = jax.lax.dot_general(
            v, expand, (((1,), (0,)), ((), ())),
            preferred_element_type=jnp.float32,
            precision=jax.lax.Precision.HIGHEST)
        out_stage[i] = jnp.where(even, vexp, x).reshape(_G, 32, 128)
        out_copy(i).start()
        return carry

    jax.lax.fori_loop(0, _N, compute, 0, unroll=False)

    def drain(i, carry):
        out_copy(i).wait()
        bulk_odd(i).wait()
        bulk_rest(i).wait()
        return carry

    jax.lax.fori_loop(0, _N, drain, 0, unroll=False)


def kernel(x_aug, values):
    x7 = x_aug.reshape(_B, _T, _G, 4, 32, 128)
    v5 = values.reshape(_B, _T2, _G, 32, 64)
    out = pl.pallas_call(
        _body,
        in_specs=[
            pl.BlockSpec(memory_space=pl.ANY),
            pl.BlockSpec(memory_space=pltpu.VMEM),
        ],
        out_specs=pl.BlockSpec(memory_space=pl.ANY),
        out_shape=jax.ShapeDtypeStruct((_B, _T, _G, 4, 32, 128), jnp.float32),
        scratch_shapes=[
            pltpu.VMEM((_N, _G, 32, 128), jnp.float32),
            pltpu.VMEM((_N, _G, 32, 128), jnp.float32),
            pltpu.SemaphoreType.DMA((_N,)),
            pltpu.SemaphoreType.DMA((_N,)),
            pltpu.SemaphoreType.DMA,
        ],
    )(x7, v5)
    return out.reshape(_B, _T, _C, _H, _W)


# speedup vs baseline: 7.8387x; 7.8387x over previous
import jax
import jax.numpy as jnp
from jax.experimental import pallas as pl
from jax.experimental.pallas import tpu as pltpu

_B, _T, _C, _H, _W = 8, 16, 64, 64, 64
_T2 = _T // 2
_G = 16
_NCHUNK = _B * _T
_NI = 12
_NO = 12


def _body(x_ref, v_ref, o_ref, in_stage, out_stage, in_sems, out_sems):
    even = (jax.lax.broadcasted_iota(jnp.int32, (_G * 32, 128), 1) % 2) == 0
    row = jax.lax.broadcasted_iota(jnp.int32, (64, 128), 0)
    col = jax.lax.broadcasted_iota(jnp.int32, (64, 128), 1)
    expand = jnp.where(col // 2 == row, 1.0, 0.0).astype(jnp.float32)

    def in_copy(j):
        s = jax.lax.rem(j, _NI)
        return pltpu.make_async_copy(x_ref.at[j], in_stage.at[s], in_sems.at[s])

    def out_copy(j):
        s = jax.lax.rem(j, _NO)
        return pltpu.make_async_copy(out_stage.at[s], o_ref.at[j], out_sems.at[s])

    def prime(j, c):
        in_copy(j).start()
        return c

    jax.lax.fori_loop(0, _NI, prime, 0, unroll=False)

    def step(j, c):
        @pl.when(j >= _NO)
        def _():
            out_copy(j - _NO).wait()

        in_copy(j).wait()
        si = jax.lax.rem(j, _NI)
        so = jax.lax.rem(j, _NO)
        out_stage[so] = in_stage[si]
        b = jax.lax.div(j, _T)
        t = jax.lax.rem(j, _T)

        @pl.when(jax.lax.rem(t, 2) == 0)
        def _():
            x = in_stage[si, :, 0].reshape(_G * 32, 128)
            v = v_ref[b, jax.lax.div(t, 2)].reshape(_G * 32, 64)
            vexp = jax.lax.dot_general(
                v, expand, (((1,), (0,)), ((), ())),
                preferred_element_type=jnp.float32,
                precision=jax.lax.Precision.HIGHEST)
            out_stage[so, :, 0] = jnp.where(even, vexp, x).reshape(_G, 32, 128)

        out_copy(j).start()

        @pl.when(j + _NI < _NCHUNK)
        def _():
            in_copy(j + _NI).start()

        return c

    jax.lax.fori_loop(0, _NCHUNK, step, 0, unroll=False)

    def drain(j, c):
        out_copy(j).wait()
        return c

    jax.lax.fori_loop(_NCHUNK - _NO, _NCHUNK, drain, 0, unroll=False)


def kernel(x_aug, values):
    x6 = x_aug.reshape(_NCHUNK, _G, 4, 32, 128)
    v5 = values.reshape(_B, _T2, _G, 32, 64)
    out = pl.pallas_call(
        _body,
        in_specs=[
            pl.BlockSpec(memory_space=pl.ANY),
            pl.BlockSpec(memory_space=pltpu.VMEM),
        ],
        out_specs=pl.BlockSpec(memory_space=pl.ANY),
        out_shape=jax.ShapeDtypeStruct((_NCHUNK, _G, 4, 32, 128), jnp.float32),
        scratch_shapes=[
            pltpu.VMEM((_NI, _G, 4, 32, 128), jnp.float32),
            pltpu.VMEM((_NO, _G, 4, 32, 128), jnp.float32),
            pltpu.SemaphoreType.DMA((_NI,)),
            pltpu.SemaphoreType.DMA((_NO,)),
        ],
    )(x6, v5)
    return out.reshape(_B, _T, _C, _H, _W)


# speedup vs baseline: 7.8524x; 1.0018x over previous
import jax
import jax.numpy as jnp
from jax.experimental import pallas as pl
from jax.experimental.pallas import tpu as pltpu

_B, _T, _C, _H, _W = 8, 16, 64, 64, 64
_T2 = _T // 2
_G = 16
_NCHUNK = _B * _T
_NS = 24
_D = 16
_U = 8


def _body(x_ref, v_ref, o_ref, buf, in_sems, out_sems):
    even = (jax.lax.broadcasted_iota(jnp.int32, (_G * 32, 128), 1) % 2) == 0
    row = jax.lax.broadcasted_iota(jnp.int32, (64, 128), 0)
    col = jax.lax.broadcasted_iota(jnp.int32, (64, 128), 1)
    expand = jnp.where(col // 2 == row, 1.0, 0.0).astype(jnp.float32)

    def in_copy(j):
        s = jax.lax.rem(j, _NS)
        return pltpu.make_async_copy(x_ref.at[j], buf.at[s], in_sems.at[s])

    def out_copy(j):
        s = jax.lax.rem(j, _NS)
        return pltpu.make_async_copy(buf.at[s], o_ref.at[j], out_sems.at[s])

    for m in range(_D):
        in_copy(m).start(priority=m % 2)

    def step(g, c):
        for p in range(_U):
            j = g * _U + p

            @pl.when(j >= _NS - _D)
            def _():
                out_copy(j - (_NS - _D)).wait()

            @pl.when(j + _D < _NCHUNK)
            def _(p=p, j=j):
                in_copy(j + _D).start(priority=(p + _D) % 2)

            in_copy(j).wait()
            if p % 2 == 0:
                s = jax.lax.rem(j, _NS)
                b = jax.lax.div(j, _T)
                t2 = jax.lax.div(jax.lax.rem(j, _T), 2)
                x = buf[s, :, 0].reshape(_G * 32, 128)
                v = v_ref[b, t2].reshape(_G * 32, 64)
                vexp = jax.lax.dot_general(
                    v, expand, (((1,), (0,)), ((), ())),
                    preferred_element_type=jnp.float32,
                    precision=jax.lax.Precision.HIGHEST)
                buf[s, :, 0] = jnp.where(even, vexp, x).reshape(_G, 32, 128)
            out_copy(j).start(priority=p % 2)
        return c

    jax.lax.fori_loop(0, _NCHUNK // _U, step, 0, unroll=False)

    for m in range(_NCHUNK - (_NS - _D), _NCHUNK):
        out_copy(m).wait()


def kernel(x_aug, values):
    x6 = x_aug.reshape(_NCHUNK, _G, 4, 32, 128)
    v5 = values.reshape(_B, _T2, _G, 32, 64)
    out = pl.pallas_call(
        _body,
        in_specs=[
            pl.BlockSpec(memory_space=pl.ANY),
            pl.BlockSpec(memory_space=pltpu.VMEM),
        ],
        out_specs=pl.BlockSpec(memory_space=pl.ANY),
        out_shape=jax.ShapeDtypeStruct((_NCHUNK, _G, 4, 32, 128), jnp.float32),
        scratch_shapes=[
            pltpu.VMEM((_NS, _G, 4, 32, 128), jnp.float32),
            pltpu.SemaphoreType.DMA((_NS,)),
            pltpu.SemaphoreType.DMA((_NS,)),
        ],
    )(x6, v5)
    return out.reshape(_B, _T, _C, _H, _W)


# speedup vs baseline: 20.4025x; 2.5982x over previous
import jax
import jax.numpy as jnp
from jax.experimental import pallas as pl
from jax.experimental.pallas import tpu as pltpu

_B, _T, _C, _H, _W = 8, 16, 64, 64, 64
_T2 = _T // 2
_G = 16
_NCHUNK = _B * _T
_NS = 20
_D = 10
_U = 8


def _body(x_ref, v_ref, o_ref, buf, in_sems, out_sems):
    even = (jax.lax.broadcasted_iota(jnp.int32, (_G, 16, 64), 2) % 2) == 0
    q128 = jax.lax.broadcasted_iota(jnp.int32, (128, 64), 0)
    l64 = jax.lax.broadcasted_iota(jnp.int32, (128, 64), 1)

    def in_copy(j):
        s = jax.lax.rem(j, _NS)
        b = jax.lax.div(j, _T)
        t = jax.lax.rem(j, _T)
        return pltpu.make_async_copy(x_ref.at[b, t], buf.at[s], in_sems.at[s])

    def out_copy(j):
        s = jax.lax.rem(j, _NS)
        b = jax.lax.div(j, _T)
        t = jax.lax.rem(j, _T)
        return pltpu.make_async_copy(buf.at[s], o_ref.at[b, t], out_sems.at[s])

    for m in range(_D):
        in_copy(m).start(priority=m % 2)

    def step(g, c):
        for p in range(_U):
            j = g * _U + p

            @pl.when(j >= _NS - _D)
            def _():
                out_copy(j - (_NS - _D)).wait()

            @pl.when(j + _D < _NCHUNK)
            def _(p=p, j=j):
                in_copy(j + _D).start(priority=(p + _D) % 2)

            in_copy(j).wait()
            if p % 2 == 0:
                sl = jax.lax.rem(j, _NS)
                b = jax.lax.div(j, _T)
                t2 = jax.lax.div(jax.lax.rem(j, _T), 2)
                vmat = v_ref[b, t2].reshape(_G * 16, 128)
                for s in range(4):
                    es = jnp.where(
                        (l64 % 2 == 0) & (q128 == 32 * s + l64 // 2),
                        1.0, 0.0).astype(jnp.float32)
                    ws = jax.lax.dot_general(
                        vmat, es, (((1,), (0,)), ((), ())),
                        preferred_element_type=jnp.float32,
                        precision=jax.lax.Precision.HIGHEST)
                    ws = ws.reshape(_G, 16, 64)
                    xs = buf[sl, pl.ds(0, _G, 4), pl.ds(s, 16, 4), :]
                    buf[sl, pl.ds(0, _G, 4), pl.ds(s, 16, 4), :] = (
                        jnp.where(even, ws, xs))
            out_copy(j).start(priority=p % 2)
        return c

    jax.lax.fori_loop(0, _NCHUNK // _U, step, 0, unroll=False)

    for m in range(_NCHUNK - (_NS - _D), _NCHUNK):
        out_copy(m).wait()


def kernel(x_aug, values):
    v4 = values.reshape(_B, _T2, _G, 16, 128)
    out = pl.pallas_call(
        _body,
        in_specs=[
            pl.BlockSpec(memory_space=pl.ANY),
            pl.BlockSpec(memory_space=pltpu.VMEM),
        ],
        out_specs=pl.BlockSpec(memory_space=pl.ANY),
        out_shape=jax.ShapeDtypeStruct((_B, _T, _C, _H, _W), jnp.float32),
        scratch_shapes=[
            pltpu.VMEM((_NS, _C, _H, _W), jnp.float32),
            pltpu.SemaphoreType.DMA((_NS,)),
            pltpu.SemaphoreType.DMA((_NS,)),
        ],
    )(x_aug, v4)
    return out


# speedup vs baseline: 20.4431x; 1.0020x over previous
import jax
import jax.numpy as jnp
from jax.experimental import pallas as pl
from jax.experimental.pallas import tpu as pltpu

_B, _T, _C, _H, _W = 8, 16, 64, 64, 64
_T2 = _T // 2
_G = 16
_NCHUNK = _B * _T
_NS = 22
_D = 11
_U = 8


def _body(x_ref, v_ref, o_ref, buf, in_sems, out_sems):
    even = (jax.lax.broadcasted_iota(jnp.int32, (_G, 16, 64), 2) % 2) == 0
    q128 = jax.lax.broadcasted_iota(jnp.int32, (128, 64), 0)
    l64 = jax.lax.broadcasted_iota(jnp.int32, (128, 64), 1)

    def in_copy(j):
        s = jax.lax.rem(j, _NS)
        b = jax.lax.div(j, _T)
        t = jax.lax.rem(j, _T)
        return pltpu.make_async_copy(x_ref.at[b, t], buf.at[s], in_sems.at[s])

    def out_copy(j):
        s = jax.lax.rem(j, _NS)
        b = jax.lax.div(j, _T)
        t = jax.lax.rem(j, _T)
        return pltpu.make_async_copy(buf.at[s], o_ref.at[b, t], out_sems.at[s])

    for m in range(_D):
        in_copy(m).start(priority=m % 2)

    def step(g, c):
        for p in range(_U):
            j = g * _U + p

            @pl.when(j >= _NS - _D)
            def _():
                out_copy(j - (_NS - _D)).wait()

            @pl.when(j + _D < _NCHUNK)
            def _(p=p, j=j):
                in_copy(j + _D).start(priority=(p + _D) % 2)

            in_copy(j).wait()
            if p % 2 == 0:
                sl = jax.lax.rem(j, _NS)
                b = jax.lax.div(j, _T)
                t2 = jax.lax.div(jax.lax.rem(j, _T), 2)
                vmat = v_ref[b, t2].reshape(_G * 16, 128)
                for s in range(4):
                    es = jnp.where(
                        (l64 % 2 == 0) & (q128 == 32 * s + l64 // 2),
                        1.0, 0.0).astype(jnp.float32)
                    ws = jax.lax.dot_general(
                        vmat, es, (((1,), (0,)), ((), ())),
                        preferred_element_type=jnp.float32,
                        precision=jax.lax.Precision.HIGHEST)
                    ws = ws.reshape(_G, 16, 64)
                    xs = buf[sl, pl.ds(0, _G, 4), pl.ds(s, 16, 4), :]
                    buf[sl, pl.ds(0, _G, 4), pl.ds(s, 16, 4), :] = (
                        jnp.where(even, ws, xs))
            out_copy(j).start(priority=p % 2)
        return c

    jax.lax.fori_loop(0, _NCHUNK // _U, step, 0, unroll=False)

    for m in range(_NCHUNK - (_NS - _D), _NCHUNK):
        out_copy(m).wait()


def kernel(x_aug, values):
    v4 = values.reshape(_B, _T2, _G, 16, 128)
    out = pl.pallas_call(
        _body,
        in_specs=[
            pl.BlockSpec(memory_space=pl.ANY),
            pl.BlockSpec(memory_space=pltpu.VMEM),
        ],
        out_specs=pl.BlockSpec(memory_space=pl.ANY),
        out_shape=jax.ShapeDtypeStruct((_B, _T, _C, _H, _W), jnp.float32),
        scratch_shapes=[
            pltpu.VMEM((_NS, _C, _H, _W), jnp.float32),
            pltpu.SemaphoreType.DMA((_NS,)),
            pltpu.SemaphoreType.DMA((_NS,)),
        ],
    )(x_aug, v4)
    return out
